# Initial kernel scaffold; baseline (speedup 1.0000x reference)
import jax
import jax.numpy as jnp
from jax.experimental import pallas as pl


def kernel(x):
    raise NotImplementedError("write your pallas kernel here")


# speedup vs baseline: 2.2627x; 2.2627x over previous
import functools

import jax
import jax.numpy as jnp
from jax.experimental import pallas as pl

_ANCHOR_W = (10.0, 16.0, 33.0)
_ANCHOR_H = (13.0, 30.0, 23.0)
_NUM_ATTRS = 85


def _decode_body(x_ref, o_ref, *, gs, stride):
    s = gs * gs
    a = pl.program_id(1)
    xb = x_ref[0, 0]

    tx = xb[0:1, :]
    ty = xb[1:2, :]
    tw = xb[2:3, :]
    th = xb[3:4, :]
    conf = xb[4:5, :]
    cls = xb[5:, :]

    k = jax.lax.broadcasted_iota(jnp.int32, (1, s), 1)
    gx = (k % gs).astype(jnp.float32)
    gy = (k // gs).astype(jnp.float32)

    aw = jnp.where(a == 0, _ANCHOR_W[0], jnp.where(a == 1, _ANCHOR_W[1], _ANCHOR_W[2]))
    ah = jnp.where(a == 0, _ANCHOR_H[0], jnp.where(a == 1, _ANCHOR_H[1], _ANCHOR_H[2]))

    bx = (jax.nn.sigmoid(tx) + gx) * stride
    by = (jax.nn.sigmoid(ty) + gy) * stride
    bw = jnp.exp(tw) * aw
    bh = jnp.exp(th) * ah
    pc = jax.nn.sigmoid(conf)

    m = jnp.max(cls, axis=0, keepdims=True)
    e = jnp.exp(cls - m)
    sm = e / jnp.sum(e, axis=0, keepdims=True)

    res = jnp.concatenate([bx, by, bw, bh, pc, sm], axis=0)
    o_ref[0, 0] = res.T


def kernel(x):
    bs, ch, gs, _ = x.shape
    nA = len(_ANCHOR_W)
    s = gs * gs
    stride = 608 // gs
    xr = x.reshape(bs, nA, _NUM_ATTRS, s)
    out = pl.pallas_call(
        functools.partial(_decode_body, gs=gs, stride=float(stride)),
        grid=(bs, nA),
        in_specs=[pl.BlockSpec((1, 1, _NUM_ATTRS, s), lambda b, a: (b, a, 0, 0))],
        out_specs=pl.BlockSpec((1, 1, s, _NUM_ATTRS), lambda b, a: (b, a, 0, 0)),
        out_shape=jax.ShapeDtypeStruct((bs, nA, s, _NUM_ATTRS), jnp.float32),
    )(xr)
    return out.reshape(bs, nA * s, _NUM_ATTRS)


# speedup vs baseline: 4.0597x; 1.7942x over previous
import functools

import jax
import jax.numpy as jnp
from jax.experimental import pallas as pl

_ANCHOR_W = (10.0, 16.0, 33.0)
_ANCHOR_H = (13.0, 30.0, 23.0)
_NUM_ATTRS = 85


def _decode_body(x_ref, o_ref, *, gs, stride):
    s = gs * gs
    a = pl.program_id(1)
    xb = x_ref[0].reshape(_NUM_ATTRS, s)

    tx = xb[0:1, :]
    ty = xb[1:2, :]
    tw = xb[2:3, :]
    th = xb[3:4, :]
    conf = xb[4:5, :]
    cls = xb[5:, :]

    k = jax.lax.broadcasted_iota(jnp.int32, (1, s), 1)
    gx = (k % gs).astype(jnp.float32)
    gy = (k // gs).astype(jnp.float32)

    aw = jnp.where(a == 0, _ANCHOR_W[0], jnp.where(a == 1, _ANCHOR_W[1], _ANCHOR_W[2]))
    ah = jnp.where(a == 0, _ANCHOR_H[0], jnp.where(a == 1, _ANCHOR_H[1], _ANCHOR_H[2]))

    bx = (jax.nn.sigmoid(tx) + gx) * stride
    by = (jax.nn.sigmoid(ty) + gy) * stride
    bw = jnp.exp(tw) * aw
    bh = jnp.exp(th) * ah
    pc = jax.nn.sigmoid(conf)

    m = jnp.max(cls, axis=0, keepdims=True)
    e = jnp.exp(cls - m)
    sm = e / jnp.sum(e, axis=0, keepdims=True)

    res = jnp.concatenate([bx, by, bw, bh, pc, sm], axis=0)
    o_ref[0, 0] = res.T


def kernel(x):
    bs, ch, gs, _ = x.shape
    nA = len(_ANCHOR_W)
    s = gs * gs
    stride = 608 // gs
    out = pl.pallas_call(
        functools.partial(_decode_body, gs=gs, stride=float(stride)),
        grid=(bs, nA),
        in_specs=[pl.BlockSpec((1, _NUM_ATTRS, gs, gs), lambda b, a: (b, a, 0, 0))],
        out_specs=pl.BlockSpec((1, 1, s, _NUM_ATTRS), lambda b, a: (b, a, 0, 0)),
        out_shape=jax.ShapeDtypeStruct((bs, nA, s, _NUM_ATTRS), jnp.float32),
    )(x)
    return out.reshape(bs, nA * s, _NUM_ATTRS)
